# baseline (speedup 1.0000x reference)
import functools

import jax
import jax.numpy as jnp
from jax import lax
from jax.experimental import pallas as pl
from jax.experimental.pallas import tpu as pltpu
from jax.experimental.pallas import tpu_sc as plsc

_NC, _NS = 2, 16
_NW = _NC * _NS
_CH = 32 * 1024


def _sc_body(x_hbm, emb_hbm, out_hbm, xbuf, ebuf, sem):
    wid = lax.axis_index("s") * _NC + lax.axis_index("c")
    n_words = x_hbm.shape[0]
    emb_words = emb_hbm.shape[0]
    wpw = n_words // _NW
    w0 = wid * wpw
    e0 = lax.rem(w0, emb_words)
    nchunks = wpw // _CH

    def chunk(c, carry):
        base = w0 + c * _CH
        ebase = e0 + c * _CH
        pltpu.sync_copy(x_hbm.at[pl.ds(base, _CH)], xbuf)
        pltpu.sync_copy(emb_hbm.at[pl.ds(ebase, _CH)], ebuf)

        @plsc.parallel_loop(0, _CH, 16, unroll=8)
        def add(o):
            xbuf[pl.ds(o, 16)] += ebuf[pl.ds(o, 16)]

        pltpu.sync_copy(xbuf, out_hbm.at[pl.ds(base, _CH)])
        return carry

    lax.fori_loop(0, nchunks, chunk, 0)


def _sc_add(xf, embf, n_words):
    kfn = pl.kernel(
        _sc_body,
        out_type=jax.ShapeDtypeStruct((n_words,), jnp.float32),
        mesh=plsc.VectorSubcoreMesh(
            core_axis_name="c", subcore_axis_name="s",
            num_cores=_NC, num_subcores=_NS,
        ),
        scratch_types=[
            pltpu.VMEM((_CH,), jnp.float32),
            pltpu.VMEM((_CH,), jnp.float32),
            pltpu.SemaphoreType.DMA,
        ],
    )
    return kfn(xf, embf)


def kernel(x, embedding):
    batch, seq_len, d_model = x.shape
    n_words = batch * seq_len * d_model
    out = _sc_add(x.reshape(n_words), embedding.reshape(seq_len * d_model),
                  n_words)
    return out.reshape(batch, seq_len, d_model)


# speedup vs baseline: 5.6452x; 5.6452x over previous
import jax
import jax.numpy as jnp
from jax.experimental import pallas as pl


_BS = 512


def _add_kernel(x_ref, emb_ref, o_ref):
    o_ref[...] = x_ref[...] + emb_ref[...][None, :, :]


def kernel(x, embedding):
    batch, seq_len, d_model = x.shape
    bs = _BS if seq_len % _BS == 0 else seq_len
    grid = (seq_len // bs,)
    return pl.pallas_call(
        _add_kernel,
        grid=grid,
        in_specs=[
            pl.BlockSpec((batch, bs, d_model), lambda s: (0, s, 0)),
            pl.BlockSpec((bs, d_model), lambda s: (s, 0)),
        ],
        out_specs=pl.BlockSpec((batch, bs, d_model), lambda s: (0, s, 0)),
        out_shape=jax.ShapeDtypeStruct((batch, seq_len, d_model), x.dtype),
    )(x, embedding)
